# Initial kernel scaffold; baseline (speedup 1.0000x reference)
import jax
import jax.numpy as jnp
from jax.experimental import pallas as pl


def kernel(x, lin1_w, lin1_b, lin2_w, lin2_b, temp, edge_index):
    raise NotImplementedError("write your pallas kernel here")


# speedup vs baseline: 12.7753x; 12.7753x over previous
import functools

import jax
import jax.numpy as jnp
from jax import lax
from jax.experimental import pallas as pl
from jax.experimental.pallas import tpu as pltpu
from jax.experimental.pallas import tpu_sc as plsc

N = 10000
E = 320000
D = 40
K = 10
P = 10240
NW = 32
NODES_PER_SC_TILE = P // 16
EPT = E // NW
CHUNK = 128
NCHUNK = (EPT + CHUNK - 1) // CHUNK
EPT_PAD = NCHUNK * CHUNK
PAD_NODE = N + 100

_mesh = plsc.VectorSubcoreMesh(core_axis_name="c", subcore_axis_name="s")


@functools.partial(
    pl.kernel,
    out_type=[jax.ShapeDtypeStruct((P, D), jnp.float32),
              jax.ShapeDtypeStruct((P, D), jnp.float32)],
    mesh=_mesh,
    scratch_types=[
        pltpu.VMEM((NCHUNK, CHUNK), jnp.int32),
        pltpu.VMEM((NCHUNK, CHUNK), jnp.int32),
        pltpu.VMEM((CHUNK, D), jnp.float32),
        pltpu.VMEM_SHARED((P, D), jnp.float32),
        pltpu.SemaphoreType.DMA,
    ],
    compiler_params=pltpu.CompilerParams(use_tc_tiling_on_sc=False),
)
def _sc_scatter(y_hbm, src_hbm, dst_hbm, zeros_hbm, out0_hbm, out1_hbm,
                src_v, dst_v, rows_v, acc_sh, sem):
    c = lax.axis_index("c")
    s = lax.axis_index("s")
    wid = s * 2 + c
    nslice = pl.ds(s * NODES_PER_SC_TILE, NODES_PER_SC_TILE)
    pltpu.sync_copy(zeros_hbm.at[nslice], acc_sh.at[nslice])
    pltpu.sync_copy(src_hbm.at[wid], src_v)
    pltpu.sync_copy(dst_hbm.at[wid], dst_v)
    plsc.subcore_barrier()

    def body(j, carry):
        pltpu.async_copy(y_hbm.at[src_v.at[j]], rows_v, sem).wait()
        pltpu.sync_copy(rows_v, acc_sh.at[dst_v.at[j]], add=True)
        return carry

    lax.fori_loop(0, NCHUNK, body, 0)
    plsc.subcore_barrier()

    @pl.when(c == 0)
    def _():
        pltpu.sync_copy(acc_sh.at[nslice], out0_hbm.at[nslice])

    @pl.when(c == 1)
    def _():
        pltpu.sync_copy(acc_sh.at[nslice], out1_hbm.at[nslice])


_BLK = 2048
_GRID = P // _BLK


def _mlp_body(x_ref, w1_ref, b1_ref, w2_ref, b2_ref, o_ref):
    x = x_ref[...]
    h1 = lax.dot_general(x, w1_ref[...], (((1,), (1,)), ((), ())),
                         preferred_element_type=jnp.float32)
    h1 = jax.nn.relu(h1 + b1_ref[...][None, :])
    h2 = lax.dot_general(h1, w2_ref[...], (((1,), (1,)), ((), ())),
                         preferred_element_type=jnp.float32)
    h2 = h2 + b2_ref[...][None, :]
    row = pl.program_id(0) * _BLK + lax.broadcasted_iota(jnp.int32, h2.shape, 0)
    o_ref[...] = jnp.where(row < N, h2, 0.0)


def _mlp(xp, w1, b1, w2, b2):
    return pl.pallas_call(
        _mlp_body,
        grid=(_GRID,),
        in_specs=[
            pl.BlockSpec((_BLK, 128), lambda i: (i, 0)),
            pl.BlockSpec((128, 128), lambda i: (0, 0)),
            pl.BlockSpec((128,), lambda i: (0,)),
            pl.BlockSpec((D, 128), lambda i: (0, 0)),
            pl.BlockSpec((D,), lambda i: (0,)),
        ],
        out_specs=pl.BlockSpec((_BLK, D), lambda i: (i, 0)),
        out_shape=jax.ShapeDtypeStruct((P, D), jnp.float32),
    )(xp, w1, b1, w2, b2)


def _pre_body(pd0_ref, pd1_ref, h_ref, o_y, o_dis, o_dis2):
    deg = pd0_ref[:, 0:1] + pd1_ref[:, 0:1] + 1.0
    dis = lax.rsqrt(deg)
    dis2 = 1.0 / deg
    o_y[...] = dis * h_ref[...]
    o_dis[...] = jnp.broadcast_to(dis, o_dis.shape)
    o_dis2[...] = jnp.broadcast_to(dis2, o_dis2.shape)


def _pre(pd0, pd1, h):
    spec = pl.BlockSpec((_BLK, D), lambda i: (i, 0))
    return pl.pallas_call(
        _pre_body,
        grid=(_GRID,),
        in_specs=[spec, spec, spec],
        out_specs=[spec, spec, spec],
        out_shape=[jax.ShapeDtypeStruct((P, D), jnp.float32)] * 3,
    )(pd0, pd1, h)


def _combine_body(tj_ref, p0_ref, p1_ref, y_ref, s_ref, dis2_ref, o_y, o_s):
    z = p0_ref[...] + p1_ref[...] + y_ref[...]
    o_s[...] = s_ref[...] + tj_ref[0] * z
    o_y[...] = dis2_ref[...] * z


def _combine(tj, p0, p1, y, s, dis2):
    spec = pl.BlockSpec((_BLK, D), lambda i: (i, 0))
    return pl.pallas_call(
        _combine_body,
        grid=(_GRID,),
        in_specs=[pl.BlockSpec(memory_space=pltpu.SMEM),
                  spec, spec, spec, spec, spec],
        out_specs=[spec, spec],
        out_shape=[jax.ShapeDtypeStruct((P, D), jnp.float32)] * 2,
    )(tj, p0, p1, y, s, dis2)


def _final_body(t0_ref, h_ref, dis_ref, s_ref, o_ref):
    o_ref[...] = t0_ref[0] * h_ref[...] + dis_ref[...] * s_ref[...]


def _final(t0, h, dis, s):
    spec = pl.BlockSpec((_BLK, D), lambda i: (i, 0))
    return pl.pallas_call(
        _final_body,
        grid=(_GRID,),
        in_specs=[pl.BlockSpec(memory_space=pltpu.SMEM), spec, spec, spec],
        out_specs=spec,
        out_shape=jax.ShapeDtypeStruct((P, D), jnp.float32),
    )(t0, h, dis, s)


def kernel(x, lin1_w, lin1_b, lin2_w, lin2_b, temp, edge_index):
    xp = jnp.zeros((P, 128), jnp.float32).at[:N].set(x)
    h = _mlp(xp, lin1_w, lin1_b, lin2_w, lin2_b)

    def _tile_idx(v):
        v2 = v.reshape(NW, EPT)
        vp = jnp.full((NW, EPT_PAD), PAD_NODE, jnp.int32).at[:, :EPT].set(v2)
        return vp.reshape(NW, NCHUNK, CHUNK)

    src_p = _tile_idx(edge_index[0])
    dst_p = _tile_idx(edge_index[1])

    zeros = jnp.zeros((P, D), jnp.float32)
    ones_tbl = jnp.zeros((P, D), jnp.float32).at[:N].set(1.0)

    pd0, pd1 = _sc_scatter(ones_tbl, src_p, dst_p, zeros)
    y, dis, dis2 = _pre(pd0, pd1, h)

    s = zeros
    for k in range(K):
        p0, p1 = _sc_scatter(y, src_p, dst_p, zeros)
        y, s = _combine(temp[k + 1:k + 2], p0, p1, y, s, dis2)

    out = _final(temp[0:1], h, dis, s)
    return out[:N]
